# baseline (speedup 1.0000x reference)
import jax
import jax.numpy as jnp
from jax import lax
from jax.experimental import pallas as pl
from jax.experimental.pallas import tpu as pltpu
from jax.experimental.pallas import tpu_sc as plsc

NC = 2
NS = 16
BK = 128
NBUF = 6


def _nan_guard(v):
    return jnp.nan_to_num(v, nan=0.0)


def kernel(x, edge_index, W, b, gamma, beta):
    N, D = x.shape
    E = edge_index.shape[1]
    DH = D // NC
    EPW = -(-E // NS)
    NB = -(-EPW // BK)
    NB = -(-NB // NBUF) * NBUF
    EPAD = NS * NB * BK
    NP = (-(-(N + 1) // BK)) * BK
    NPT = NP // NS
    assert NPT % 8 == 0 and D % NC == 0

    src = edge_index[0].astype(jnp.int32)
    dst = edge_index[1].astype(jnp.int32)
    pad = EPAD - E
    src3 = jnp.concatenate([src, jnp.zeros((pad,), jnp.int32)]).reshape(NS, NB, BK)
    dst3 = jnp.concatenate([dst, jnp.full((pad,), N, jnp.int32)]).reshape(NS, NB, BK)

    zrow = jnp.zeros((NPT,), jnp.float32)
    zacc = jnp.zeros((BK, DH), jnp.float32)
    ones = jnp.ones((BK,), jnp.float32)

    mesh = plsc.VectorSubcoreMesh(core_axis_name="c", subcore_axis_name="s")
    sc_params = pltpu.CompilerParams(use_tc_tiling_on_sc=False)

    def deg_body(dst_hbm, ones_hbm, zrow_hbm, deg_hbm, idx_v, ones_v, stage_v,
                 dsem, acc_sh):
        c = lax.axis_index("c")
        s = lax.axis_index("s")
        pltpu.sync_copy(zrow_hbm, stage_v)
        pltpu.sync_copy(stage_v, acc_sh.at[pl.ds(s * NPT, NPT)])
        pltpu.sync_copy(ones_hbm, ones_v)
        pltpu.sync_copy(dst_hbm.at[s], idx_v)
        plsc.subcore_barrier()

        def blk(j, carry):
            pltpu.async_copy(ones_v, acc_sh.at[idx_v.at[j]], dsem, add=True)
            return carry

        lax.fori_loop(0, NB, blk, 0)

        def drain(j, carry):
            pltpu.make_async_copy(ones_v, acc_sh.at[idx_v.at[0]], dsem).wait()
            return carry

        lax.fori_loop(0, NB, drain, 0)
        plsc.subcore_barrier()
        pltpu.sync_copy(acc_sh.at[pl.ds(s * NPT, NPT)], stage_v)
        pltpu.sync_copy(stage_v, deg_hbm.at[pl.ds(c * NP + s * NPT, NPT)])

    deg_1d = pl.kernel(
        deg_body,
        out_type=jax.ShapeDtypeStruct((NC * NP,), jnp.float32),
        mesh=mesh,
        compiler_params=sc_params,
        scratch_types=[
            pltpu.VMEM((NB, BK), jnp.int32),
            pltpu.VMEM((BK,), jnp.float32),
            pltpu.VMEM((NPT,), jnp.float32),
            pltpu.SemaphoreType.DMA,
            pltpu.VMEM_SHARED((NP,), jnp.float32),
        ],
    )(dst3, ones, zrow)
    dcol = deg_1d[:NP].reshape(NP, 1)

    xp = jnp.pad(x, ((0, NP - N), (0, 0)))
    Wh = jnp.stack([W[:, h * DH:(h + 1) * DH] for h in range(NC)])
    BM = NP // 8

    def mm_body(x_ref, w_ref, dc_ref, yw_ref, dv_ref):
        dv = lax.rsqrt(dc_ref[...] + 1.0)
        dv_ref[...] = dv
        yw_ref[0] = jnp.dot(_nan_guard(x_ref[...]), w_ref[0],
                            preferred_element_type=jnp.float32) * dv

    yw2, dinv = pl.pallas_call(
        mm_body,
        grid=(NP // BM, NC),
        in_specs=[
            pl.BlockSpec((BM, D), lambda i, h: (i, 0)),
            pl.BlockSpec((1, D, DH), lambda i, h: (h, 0, 0)),
            pl.BlockSpec((BM, 1), lambda i, h: (i, 0)),
        ],
        out_specs=[
            pl.BlockSpec((1, BM, DH), lambda i, h: (h, i, 0)),
            pl.BlockSpec((BM, 1), lambda i, h: (i, 0)),
        ],
        out_shape=[
            jax.ShapeDtypeStruct((NC, NP, DH), jnp.float32),
            jax.ShapeDtypeStruct((NP, 1), jnp.float32),
        ],
    )(xp, Wh, dcol)

    NKF = NPT // BK
    TAIL = NPT - NKF * BK
    LA = NBUF - 1

    def agg_body(src_hbm, dst_hbm, yw_hbm, zacc_hbm, acc_hbm,
                 sidx_v, didx_v, r0, r1, r2, r3, r4, r5,
                 gsem, ssem, acc_sh):
        rows = (r0, r1, r2, r3, r4, r5)
        c = lax.axis_index("c")
        s = lax.axis_index("s")
        ywc = yw_hbm.at[c]
        pltpu.sync_copy(zacc_hbm, r0)
        for k in range(NKF):
            pltpu.sync_copy(r0, acc_sh.at[pl.ds(s * NPT + k * BK, BK)])
        if TAIL:
            pltpu.sync_copy(r0.at[pl.ds(0, TAIL)],
                            acc_sh.at[pl.ds(s * NPT + NKF * BK, TAIL)])
        pltpu.sync_copy(src_hbm.at[s], sidx_v)
        pltpu.sync_copy(dst_hbm.at[s], didx_v)
        plsc.subcore_barrier()

        def fire_gather(blk, bf):
            pltpu.async_copy(ywc.at[sidx_v.at[blk]], rows[bf], gsem.at[bf])

        def wait_gather(bf):
            pltpu.make_async_copy(ywc.at[sidx_v.at[0]], rows[bf],
                                  gsem.at[bf]).wait()

        def fire_scatter(blk, bf):
            pltpu.async_copy(rows[bf], acc_sh.at[didx_v.at[blk]], ssem.at[bf],
                             add=True)

        def wait_scatter(bf):
            pltpu.make_async_copy(rows[bf], acc_sh.at[didx_v.at[0]],
                                  ssem.at[bf]).wait()

        for i in range(LA):
            fire_gather(i, i)

        @pl.loop(0, NB, step=NBUF)
        def _pipe(j):
            for i in range(NBUF):
                blk = j + i
                nxt = blk + LA
                nb_i = (i + LA) % NBUF

                @pl.when(jnp.logical_and(nxt < NB, nxt >= NBUF))
                def _():
                    wait_scatter(nb_i)

                @pl.when(nxt < NB)
                def _():
                    fire_gather(nxt, nb_i)

                wait_gather(i)
                fire_scatter(blk, i)

        for bf in range(NBUF):
            wait_scatter(bf)
        plsc.subcore_barrier()
        for k in range(NKF):
            pltpu.sync_copy(acc_sh.at[pl.ds(s * NPT + k * BK, BK)], r0)
            pltpu.sync_copy(r0, acc_hbm.at[c, pl.ds(s * NPT + k * BK, BK)])
        if TAIL:
            pltpu.sync_copy(acc_sh.at[pl.ds(s * NPT + NKF * BK, TAIL)],
                            r0.at[pl.ds(0, TAIL)])
            pltpu.sync_copy(r0.at[pl.ds(0, TAIL)],
                            acc_hbm.at[c, pl.ds(s * NPT + NKF * BK, TAIL)])

    acc = pl.kernel(
        agg_body,
        out_type=jax.ShapeDtypeStruct((NC, NP, DH), jnp.float32),
        mesh=mesh,
        compiler_params=sc_params,
        scratch_types=[
            pltpu.VMEM((NB, BK), jnp.int32),
            pltpu.VMEM((NB, BK), jnp.int32),
            pltpu.VMEM((BK, DH), jnp.float32),
            pltpu.VMEM((BK, DH), jnp.float32),
            pltpu.VMEM((BK, DH), jnp.float32),
            pltpu.VMEM((BK, DH), jnp.float32),
            pltpu.VMEM((BK, DH), jnp.float32),
            pltpu.VMEM((BK, DH), jnp.float32),
            pltpu.SemaphoreType.DMA((NBUF,)),
            pltpu.SemaphoreType.DMA((NBUF,)),
            pltpu.VMEM_SHARED((NP, DH), jnp.float32),
        ],
    )(src3, dst3, yw2, zacc)

    b2 = b.reshape(1, D)
    g2 = gamma.reshape(1, D)
    be2 = beta.reshape(1, D)

    def fin_body(a_ref, yw_ref, dv_ref, x_ref, b_ref, g_ref, be_ref, o_ref):
        sh = [a_ref[h, :N] + yw_ref[h, :N] for h in range(NC)]
        s = jnp.concatenate(sh, axis=1)
        pre = s * dv_ref[:N] + b_ref[...]
        m = jnp.mean(pre, axis=0, keepdims=True)
        v = jnp.mean((pre - m) * (pre - m), axis=0, keepdims=True)
        o = (pre - m) * (g_ref[...] * lax.rsqrt(v + 1e-5)) + be_ref[...]
        o = jnp.maximum(o, 0.0) + _nan_guard(x_ref[...])
        o_ref[...] = _nan_guard(o)

    out = pl.pallas_call(
        fin_body,
        out_shape=jax.ShapeDtypeStruct((N, D), jnp.float32),
    )(acc, yw2, dinv, x, b2, g2, be2)

    return out


# speedup vs baseline: 1.4664x; 1.4664x over previous
import jax
import jax.numpy as jnp
from jax import lax
from jax.experimental import pallas as pl
from jax.experimental.pallas import tpu as pltpu
from jax.experimental.pallas import tpu_sc as plsc

NC = 2
NS = 16
BK = 128
NBUF = 4


def _nan_guard(v):
    return jnp.nan_to_num(v, nan=0.0)


def kernel(x, edge_index, W, b, gamma, beta):
    N, D = x.shape
    E = edge_index.shape[1]
    DH = D // NC
    EPW = -(-E // NS)
    NB = -(-EPW // BK)
    NB = -(-NB // NBUF) * NBUF
    EPAD = NS * NB * BK
    NP = (-(-(N + 1) // BK)) * BK
    NPT = NP // NS
    assert NPT % 8 == 0 and D % NC == 0

    src = edge_index[0].astype(jnp.int32)
    dst = edge_index[1].astype(jnp.int32)
    pad = EPAD - E
    src3 = jnp.concatenate([src, jnp.zeros((pad,), jnp.int32)]).reshape(NS, NB, BK)
    dst3 = jnp.concatenate([dst, jnp.full((pad,), N, jnp.int32)]).reshape(NS, NB, BK)

    zrow = jnp.zeros((NPT,), jnp.float32)
    zacc = jnp.zeros((BK, DH), jnp.float32)
    ones = jnp.ones((BK,), jnp.float32)

    mesh = plsc.VectorSubcoreMesh(core_axis_name="c", subcore_axis_name="s")
    sc_params = pltpu.CompilerParams(use_tc_tiling_on_sc=False,
                                     needs_layout_passes=False)

    def deg_body(dst_hbm, ones_hbm, zrow_hbm, deg_hbm, idx_v, ones_v, stage_v,
                 dsem, acc_sh):
        c = lax.axis_index("c")
        s = lax.axis_index("s")
        pltpu.sync_copy(zrow_hbm, stage_v)
        pltpu.sync_copy(stage_v, acc_sh.at[pl.ds(s * NPT, NPT)])
        pltpu.sync_copy(ones_hbm, ones_v)
        pltpu.sync_copy(dst_hbm.at[s], idx_v)
        plsc.subcore_barrier()

        def blk(j, carry):
            pltpu.async_copy(ones_v, acc_sh.at[idx_v.at[j]], dsem, add=True)
            return carry

        lax.fori_loop(0, NB, blk, 0)

        def drain(j, carry):
            pltpu.make_async_copy(ones_v, acc_sh.at[idx_v.at[0]], dsem).wait()
            return carry

        lax.fori_loop(0, NB, drain, 0)
        plsc.subcore_barrier()
        pltpu.sync_copy(acc_sh.at[pl.ds(s * NPT, NPT)], stage_v)
        pltpu.sync_copy(stage_v, deg_hbm.at[pl.ds(c * NP + s * NPT, NPT)])

    deg_1d = pl.kernel(
        deg_body,
        out_type=jax.ShapeDtypeStruct((NC * NP,), jnp.float32),
        mesh=mesh,
        compiler_params=sc_params,
        scratch_types=[
            pltpu.VMEM((NB, BK), jnp.int32),
            pltpu.VMEM((BK,), jnp.float32),
            pltpu.VMEM((NPT,), jnp.float32),
            pltpu.SemaphoreType.DMA,
            pltpu.VMEM_SHARED((NP,), jnp.float32),
        ],
    )(dst3, ones, zrow)
    dcol = deg_1d[:NP].reshape(NP, 1)

    xp = jnp.pad(x, ((0, NP - N), (0, 0)))
    Wh = jnp.stack([W[:, h * DH:(h + 1) * DH] for h in range(NC)])
    BM = NP // 8

    def mm_body(x_ref, w_ref, dc_ref, yw_ref, ywb_ref, dv_ref):
        dv = lax.rsqrt(dc_ref[...] + 1.0)
        dv_ref[...] = dv
        yw = jnp.dot(_nan_guard(x_ref[...]), w_ref[0],
                     preferred_element_type=jnp.float32) * dv
        yw_ref[0] = yw
        ywb_ref[0] = yw.astype(jnp.bfloat16)

    yw2, ywb, dinv = pl.pallas_call(
        mm_body,
        grid=(NP // BM, NC),
        in_specs=[
            pl.BlockSpec((BM, D), lambda i, h: (i, 0)),
            pl.BlockSpec((1, D, DH), lambda i, h: (h, 0, 0)),
            pl.BlockSpec((BM, 1), lambda i, h: (i, 0)),
        ],
        out_specs=[
            pl.BlockSpec((1, BM, DH), lambda i, h: (h, i, 0)),
            pl.BlockSpec((1, BM, DH), lambda i, h: (h, i, 0)),
            pl.BlockSpec((BM, 1), lambda i, h: (i, 0)),
        ],
        out_shape=[
            jax.ShapeDtypeStruct((NC, NP, DH), jnp.float32),
            jax.ShapeDtypeStruct((NC, NP, DH), jnp.bfloat16),
            jax.ShapeDtypeStruct((NP, 1), jnp.float32),
        ],
    )(xp, Wh, dcol)
    ywbp = (ywb.reshape(NC, NP, DH // 32, 2, 16)
            .swapaxes(3, 4).reshape(NC, NP, DH))

    NKF = NPT // BK
    TAIL = NPT - NKF * BK
    LA = NBUF - 1

    def agg_body(src_hbm, dst_hbm, yw_hbm, zacc_hbm, acc_hbm,
                 sidx_v, didx_v, rb0, rb1, rb2, rb3, r0, r1, r2, r3,
                 gsem, ssem, acc_sh):
        rbs = (rb0, rb1, rb2, rb3)
        rfs = (r0, r1, r2, r3)
        c = lax.axis_index("c")
        s = lax.axis_index("s")
        ywc = yw_hbm.at[c]
        pltpu.sync_copy(zacc_hbm, r0)
        for k in range(NKF):
            pltpu.sync_copy(r0, acc_sh.at[pl.ds(s * NPT + k * BK, BK)])
        if TAIL:
            pltpu.sync_copy(r0.at[pl.ds(0, TAIL)],
                            acc_sh.at[pl.ds(s * NPT + NKF * BK, TAIL)])
        pltpu.sync_copy(src_hbm.at[s], sidx_v)
        pltpu.sync_copy(dst_hbm.at[s], didx_v)
        plsc.subcore_barrier()

        def fire_gather(blk, bf):
            pltpu.async_copy(ywc.at[sidx_v.at[blk]], rbs[bf], gsem.at[bf])

        def wait_gather(bf):
            pltpu.make_async_copy(ywc.at[sidx_v.at[0]], rbs[bf],
                                  gsem.at[bf]).wait()

        def fire_scatter(blk, bf):
            pltpu.async_copy(rfs[bf], acc_sh.at[didx_v.at[blk]], ssem.at[bf],
                             add=True)

        def wait_scatter(bf):
            pltpu.make_async_copy(rfs[bf], acc_sh.at[didx_v.at[0]],
                                  ssem.at[bf]).wait()

        def convert(bf):
            def row(k, carry):
                for g in range(DH // 32):
                    v = rbs[bf][k, pl.ds(32 * g, 32)]
                    lo, hi = plsc.unpack(v, format=plsc.PackFormat.INTERLEAVED)
                    rfs[bf][k, pl.ds(32 * g, 16)] = lo
                    rfs[bf][k, pl.ds(32 * g + 16, 16)] = hi
                return carry
            lax.fori_loop(0, BK, row, 0, unroll=4)

        for i in range(LA):
            fire_gather(i, i)

        @pl.loop(0, NB, step=NBUF)
        def _pipe(j):
            for i in range(NBUF):
                blk = j + i
                nxt = blk + LA
                nb_i = (i + LA) % NBUF

                @pl.when(nxt < NB)
                def _():
                    fire_gather(nxt, nb_i)

                wait_gather(i)

                @pl.when(blk >= NBUF)
                def _():
                    wait_scatter(i)

                convert(i)
                fire_scatter(blk, i)

        for bf in range(NBUF):
            wait_scatter(bf)
        plsc.subcore_barrier()
        for k in range(NKF):
            pltpu.sync_copy(acc_sh.at[pl.ds(s * NPT + k * BK, BK)], r0)
            pltpu.sync_copy(r0, acc_hbm.at[c, pl.ds(s * NPT + k * BK, BK)])
        if TAIL:
            pltpu.sync_copy(acc_sh.at[pl.ds(s * NPT + NKF * BK, TAIL)],
                            r0.at[pl.ds(0, TAIL)])
            pltpu.sync_copy(r0.at[pl.ds(0, TAIL)],
                            acc_hbm.at[c, pl.ds(s * NPT + NKF * BK, TAIL)])

    acc = pl.kernel(
        agg_body,
        out_type=jax.ShapeDtypeStruct((NC, NP, DH), jnp.float32),
        mesh=mesh,
        compiler_params=sc_params,
        scratch_types=[
            pltpu.VMEM((NB, BK), jnp.int32),
            pltpu.VMEM((NB, BK), jnp.int32),
            pltpu.VMEM((BK, DH), jnp.bfloat16),
            pltpu.VMEM((BK, DH), jnp.bfloat16),
            pltpu.VMEM((BK, DH), jnp.bfloat16),
            pltpu.VMEM((BK, DH), jnp.bfloat16),
            pltpu.VMEM((BK, DH), jnp.float32),
            pltpu.VMEM((BK, DH), jnp.float32),
            pltpu.VMEM((BK, DH), jnp.float32),
            pltpu.VMEM((BK, DH), jnp.float32),
            pltpu.SemaphoreType.DMA((NBUF,)),
            pltpu.SemaphoreType.DMA((NBUF,)),
            pltpu.VMEM_SHARED((NP, DH), jnp.float32),
        ],
    )(src3, dst3, ywbp, zacc)

    b2 = b.reshape(1, D)
    g2 = gamma.reshape(1, D)
    be2 = beta.reshape(1, D)

    def fin_body(a_ref, yw_ref, dv_ref, x_ref, b_ref, g_ref, be_ref, o_ref):
        sh = [a_ref[h, :N] + yw_ref[h, :N] for h in range(NC)]
        s = jnp.concatenate(sh, axis=1)
        pre = s * dv_ref[:N] + b_ref[...]
        m = jnp.mean(pre, axis=0, keepdims=True)
        v = jnp.mean((pre - m) * (pre - m), axis=0, keepdims=True)
        o = (pre - m) * (g_ref[...] * lax.rsqrt(v + 1e-5)) + be_ref[...]
        o = jnp.maximum(o, 0.0) + _nan_guard(x_ref[...])
        o_ref[...] = _nan_guard(o)

    out = pl.pallas_call(
        fin_body,
        out_shape=jax.ShapeDtypeStruct((N, D), jnp.float32),
    )(acc, yw2, dinv, x, b2, g2, be2)

    return out


# speedup vs baseline: 1.5143x; 1.0326x over previous
import jax
import jax.numpy as jnp
from jax import lax
from jax.experimental import pallas as pl
from jax.experimental.pallas import tpu as pltpu
from jax.experimental.pallas import tpu_sc as plsc

NC = 2
NS = 16
BK = 128
NBUF = 4


def _nan_guard(v):
    return jnp.nan_to_num(v, nan=0.0)


def kernel(x, edge_index, W, b, gamma, beta):
    N, D = x.shape
    E = edge_index.shape[1]
    DH = D // NC
    EPW = -(-E // NS)
    NB = -(-EPW // BK)
    NB = -(-NB // NBUF) * NBUF
    EPAD = NS * NB * BK
    NP = (-(-(N + 1) // BK)) * BK
    NPT = NP // NS
    assert NPT % 8 == 0 and D % NC == 0

    src = edge_index[0].astype(jnp.int32)
    dst = edge_index[1].astype(jnp.int32)
    pad = EPAD - E
    src3 = jnp.concatenate([src, jnp.zeros((pad,), jnp.int32)]).reshape(NS, NB, BK)
    dst3 = jnp.concatenate([dst, jnp.full((pad,), N, jnp.int32)]).reshape(NS, NB, BK)

    zrow = jnp.zeros((NPT,), jnp.float32)
    zacc = jnp.zeros((BK, DH), jnp.float32)
    ones = jnp.ones((BK,), jnp.float32)

    mesh = plsc.VectorSubcoreMesh(core_axis_name="c", subcore_axis_name="s")
    sc_params = pltpu.CompilerParams(use_tc_tiling_on_sc=False,
                                     needs_layout_passes=False)

    def deg_body(dst_hbm, ones_hbm, zrow_hbm, deg_hbm, idx_v, ones_v, stage_v,
                 dsem, acc_sh):
        c = lax.axis_index("c")
        s = lax.axis_index("s")
        pltpu.sync_copy(zrow_hbm, stage_v)
        pltpu.sync_copy(stage_v, acc_sh.at[pl.ds(s * NPT, NPT)])
        pltpu.sync_copy(ones_hbm, ones_v)
        pltpu.sync_copy(dst_hbm.at[s], idx_v)
        plsc.subcore_barrier()

        def blk(j, carry):
            pltpu.async_copy(ones_v, acc_sh.at[idx_v.at[j]], dsem, add=True)
            return carry

        lax.fori_loop(0, NB, blk, 0)

        def drain(j, carry):
            pltpu.make_async_copy(ones_v, acc_sh.at[idx_v.at[0]], dsem).wait()
            return carry

        lax.fori_loop(0, NB, drain, 0)
        plsc.subcore_barrier()
        pltpu.sync_copy(acc_sh.at[pl.ds(s * NPT, NPT)], stage_v)
        pltpu.sync_copy(stage_v, deg_hbm.at[pl.ds(c * NP + s * NPT, NPT)])

    deg_1d = pl.kernel(
        deg_body,
        out_type=jax.ShapeDtypeStruct((NC * NP,), jnp.float32),
        mesh=mesh,
        compiler_params=sc_params,
        scratch_types=[
            pltpu.VMEM((NB, BK), jnp.int32),
            pltpu.VMEM((BK,), jnp.float32),
            pltpu.VMEM((NPT,), jnp.float32),
            pltpu.SemaphoreType.DMA,
            pltpu.VMEM_SHARED((NP,), jnp.float32),
        ],
    )(dst3, ones, zrow)
    dcol = deg_1d[:NP].reshape(NP, 1)

    xp = jnp.pad(x, ((0, NP - N), (0, 0)))
    Wh = jnp.stack([W[:, h * DH:(h + 1) * DH] for h in range(NC)])
    BM = NP // 8

    def mm_body(x_ref, w_ref, dc_ref, yw_ref, ywb_ref, dv_ref):
        dv = lax.rsqrt(dc_ref[...] + 1.0)
        dv_ref[...] = dv
        yw = jnp.dot(_nan_guard(x_ref[...]), w_ref[0],
                     preferred_element_type=jnp.float32) * dv
        yw_ref[0] = yw
        ywb_ref[0] = yw.astype(jnp.bfloat16)

    yw2, ywb, dinv = pl.pallas_call(
        mm_body,
        grid=(NP // BM, NC),
        in_specs=[
            pl.BlockSpec((BM, D), lambda i, h: (i, 0)),
            pl.BlockSpec((1, D, DH), lambda i, h: (h, 0, 0)),
            pl.BlockSpec((BM, 1), lambda i, h: (i, 0)),
        ],
        out_specs=[
            pl.BlockSpec((1, BM, DH), lambda i, h: (h, i, 0)),
            pl.BlockSpec((1, BM, DH), lambda i, h: (h, i, 0)),
            pl.BlockSpec((BM, 1), lambda i, h: (i, 0)),
        ],
        out_shape=[
            jax.ShapeDtypeStruct((NC, NP, DH), jnp.float32),
            jax.ShapeDtypeStruct((NC, NP, DH), jnp.bfloat16),
            jax.ShapeDtypeStruct((NP, 1), jnp.float32),
        ],
    )(xp, Wh, dcol)
    ywbp = (ywb.reshape(NC, NP, DH // 32, 2, 16)
            .swapaxes(3, 4).reshape(NC, NP, DH))

    NKF = NPT // BK
    TAIL = NPT - NKF * BK
    LA = NBUF - 1

    def agg_body(src_hbm, dst_hbm, yw_hbm, zacc_hbm, acc_hbm,
                 sidx_v, didx_v, rb0, rb1, rb2, rb3, r0, r1, r2, r3,
                 gsem, ssem, acc_sh):
        rbs = (rb0, rb1, rb2, rb3)
        rfs = (r0, r1, r2, r3)
        c = lax.axis_index("c")
        s = lax.axis_index("s")
        ywc = yw_hbm.at[c]
        pltpu.sync_copy(zacc_hbm, r0)
        for k in range(NKF):
            pltpu.sync_copy(r0, acc_sh.at[pl.ds(s * NPT + k * BK, BK)])
        if TAIL:
            pltpu.sync_copy(r0.at[pl.ds(0, TAIL)],
                            acc_sh.at[pl.ds(s * NPT + NKF * BK, TAIL)])
        pltpu.sync_copy(src_hbm.at[s], sidx_v)
        pltpu.sync_copy(dst_hbm.at[s], didx_v)
        plsc.subcore_barrier()

        def fire_gather(blk, bf):
            pltpu.async_copy(ywc.at[sidx_v.at[blk]], rbs[bf], gsem.at[bf])

        def wait_gather(bf):
            pltpu.make_async_copy(ywc.at[sidx_v.at[0]], rbs[bf],
                                  gsem.at[bf]).wait()

        def fire_scatter(blk, bf):
            pltpu.async_copy(rfs[bf], acc_sh.at[didx_v.at[blk]], ssem.at[bf],
                             add=True)

        def wait_scatter(bf):
            pltpu.make_async_copy(rfs[bf], acc_sh.at[didx_v.at[0]],
                                  ssem.at[bf]).wait()

        def convert(bf):
            def row(k, carry):
                for g in range(DH // 32):
                    v = rbs[bf][k, pl.ds(32 * g, 32)]
                    lo, hi = plsc.unpack(v, format=plsc.PackFormat.INTERLEAVED)
                    rfs[bf][k, pl.ds(32 * g, 16)] = lo
                    rfs[bf][k, pl.ds(32 * g + 16, 16)] = hi
                return carry
            lax.fori_loop(0, BK, row, 0, unroll=8)

        for i in range(LA):
            fire_gather(i, i)

        @pl.loop(0, NB, step=NBUF)
        def _pipe(j):
            for i in range(NBUF):
                blk = j + i
                nxt = blk + LA
                nb_i = (i + LA) % NBUF

                @pl.when(nxt < NB)
                def _():
                    fire_gather(nxt, nb_i)

                wait_gather(i)

                @pl.when(blk >= NBUF)
                def _():
                    wait_scatter(i)

                convert(i)
                fire_scatter(blk, i)

        for bf in range(NBUF):
            wait_scatter(bf)
        plsc.subcore_barrier()
        for k in range(NKF):
            pltpu.sync_copy(acc_sh.at[pl.ds(s * NPT + k * BK, BK)], r0)
            pltpu.sync_copy(r0, acc_hbm.at[c, pl.ds(s * NPT + k * BK, BK)])
        if TAIL:
            pltpu.sync_copy(acc_sh.at[pl.ds(s * NPT + NKF * BK, TAIL)],
                            r0.at[pl.ds(0, TAIL)])
            pltpu.sync_copy(r0.at[pl.ds(0, TAIL)],
                            acc_hbm.at[c, pl.ds(s * NPT + NKF * BK, TAIL)])

    acc = pl.kernel(
        agg_body,
        out_type=jax.ShapeDtypeStruct((NC, NP, DH), jnp.float32),
        mesh=mesh,
        compiler_params=sc_params,
        scratch_types=[
            pltpu.VMEM((NB, BK), jnp.int32),
            pltpu.VMEM((NB, BK), jnp.int32),
            pltpu.VMEM((BK, DH), jnp.bfloat16),
            pltpu.VMEM((BK, DH), jnp.bfloat16),
            pltpu.VMEM((BK, DH), jnp.bfloat16),
            pltpu.VMEM((BK, DH), jnp.bfloat16),
            pltpu.VMEM((BK, DH), jnp.float32),
            pltpu.VMEM((BK, DH), jnp.float32),
            pltpu.VMEM((BK, DH), jnp.float32),
            pltpu.VMEM((BK, DH), jnp.float32),
            pltpu.SemaphoreType.DMA((NBUF,)),
            pltpu.SemaphoreType.DMA((NBUF,)),
            pltpu.VMEM_SHARED((NP, DH), jnp.float32),
        ],
    )(src3, dst3, ywbp, zacc)

    b2 = b.reshape(1, D)
    g2 = gamma.reshape(1, D)
    be2 = beta.reshape(1, D)

    def fin_body(a_ref, yw_ref, dv_ref, x_ref, b_ref, g_ref, be_ref, o_ref):
        sh = [a_ref[h, :N] + yw_ref[h, :N] for h in range(NC)]
        s = jnp.concatenate(sh, axis=1)
        pre = s * dv_ref[:N] + b_ref[...]
        m = jnp.mean(pre, axis=0, keepdims=True)
        v = jnp.mean((pre - m) * (pre - m), axis=0, keepdims=True)
        o = (pre - m) * (g_ref[...] * lax.rsqrt(v + 1e-5)) + be_ref[...]
        o = jnp.maximum(o, 0.0) + _nan_guard(x_ref[...])
        o_ref[...] = _nan_guard(o)

    out = pl.pallas_call(
        fin_body,
        out_shape=jax.ShapeDtypeStruct((N, D), jnp.float32),
    )(acc, yw2, dinv, x, b2, g2, be2)

    return out
